# baseline (speedup 1.0000x reference)
import functools

import jax
import jax.numpy as jnp
from jax import lax
from jax.experimental import pallas as pl
from jax.experimental.pallas import tpu as pltpu
from jax.experimental.pallas import tpu_sc as plsc

_N_ITEMS = 1000
_BATCH = 16384
_NC, _NS, _L = 2, 16, 16
_NW = _NC * _NS
_CHUNK = 128
_ROWS_PER_W = _BATCH // (_NW * _CHUNK)


def _build():
    mesh = plsc.VectorSubcoreMesh(core_axis_name="c", subcore_axis_name="s")

    @functools.partial(
        pl.kernel,
        mesh=mesh,
        out_type=jax.ShapeDtypeStruct((_NW * _ROWS_PER_W, _CHUNK), jnp.float32),
        scratch_types=[
            pltpu.VMEM((_ROWS_PER_W, _CHUNK), jnp.int32),
            pltpu.VMEM((_ROWS_PER_W, _CHUNK), jnp.int32),
            pltpu.VMEM((_ROWS_PER_W, _CHUNK), jnp.int32),
            pltpu.VMEM((_ROWS_PER_W, _CHUNK), jnp.float32),
            pltpu.SemaphoreType.DMA,
        ],
    )
    def gather_kernel(users_hbm, items_hbm, scores_hbm, out_hbm,
                      u_v, it_v, idx_v, val_v, sem):
        wid = lax.axis_index("s") * _NC + lax.axis_index("c")
        base = wid * _ROWS_PER_W
        pltpu.sync_copy(users_hbm.at[pl.ds(base, _ROWS_PER_W)], u_v)
        pltpu.sync_copy(items_hbm.at[pl.ds(base, _ROWS_PER_W)], it_v)
        for j in range(_ROWS_PER_W):
            for i in range(_CHUNK // _L):
                s = pl.ds(i * _L, _L)
                idx_v[j, s] = u_v[j, s] * _N_ITEMS + it_v[j, s]
        copies = [
            pltpu.async_copy(scores_hbm.at[idx_v.at[j]], val_v.at[j], sem)
            for j in range(_ROWS_PER_W)
        ]
        for c in copies:
            c.wait()
        pltpu.sync_copy(val_v, out_hbm.at[pl.ds(base, _ROWS_PER_W)])

    return gather_kernel


_GATHER = _build()


def kernel(users, items, scores):
    u2 = users.astype(jnp.int32).reshape(_NW * _ROWS_PER_W, _CHUNK)
    it2 = items.astype(jnp.int32).reshape(_NW * _ROWS_PER_W, _CHUNK)
    flat = scores.reshape(-1)
    out = _GATHER(u2, it2, flat)
    return out.reshape(_BATCH)


# speedup vs baseline: 1.0017x; 1.0017x over previous
import functools

import jax
import jax.numpy as jnp
from jax import lax
from jax.experimental import pallas as pl
from jax.experimental.pallas import tpu as pltpu
from jax.experimental.pallas import tpu_sc as plsc

_N_ITEMS = 1000
_BATCH = 16384
_NC, _NS, _L = 2, 16, 16
_NW = _NC * _NS
_CHUNK = 128
_ROWS_PER_W = _BATCH // (_NW * _CHUNK)


def _build():
    mesh = plsc.VectorSubcoreMesh(core_axis_name="c", subcore_axis_name="s")

    @functools.partial(
        pl.kernel,
        mesh=mesh,
        compiler_params=pltpu.CompilerParams(needs_layout_passes=False),
        out_type=jax.ShapeDtypeStruct((_NW * _ROWS_PER_W, _CHUNK), jnp.float32),
        scratch_types=[
            pltpu.VMEM((_ROWS_PER_W, _CHUNK), jnp.int32),
            pltpu.VMEM((_ROWS_PER_W, _CHUNK), jnp.int32),
            pltpu.VMEM((_ROWS_PER_W, _CHUNK), jnp.int32),
            pltpu.VMEM((_ROWS_PER_W, _CHUNK), jnp.float32),
            pltpu.SemaphoreType.DMA,
        ],
    )
    def gather_kernel(users_hbm, items_hbm, scores_hbm, out_hbm,
                      u_v, it_v, flat_v, sel_v, sem):
        wid = lax.axis_index("s") * _NC + lax.axis_index("c")
        base = wid * _ROWS_PER_W
        pltpu.sync_copy(users_hbm.at[pl.ds(base, _ROWS_PER_W)], u_v)
        pltpu.sync_copy(items_hbm.at[pl.ds(base, _ROWS_PER_W)], it_v)
        for j in range(_ROWS_PER_W):
            for i in range(_CHUNK // _L):
                s = pl.ds(i * _L, _L)
                flat_v[j, s] = u_v[j, s] * _N_ITEMS + it_v[j, s]
        copies = [
            pltpu.async_copy(scores_hbm.at[flat_v.at[j]], sel_v.at[j], sem)
            for j in range(_ROWS_PER_W)
        ]
        for c in copies:
            c.wait()
        pltpu.sync_copy(sel_v, out_hbm.at[pl.ds(base, _ROWS_PER_W)])

    return gather_kernel


_GATHER = _build()


def kernel(users, items, scores):
    u2 = users.astype(jnp.int32).reshape(_NW * _ROWS_PER_W, _CHUNK)
    it2 = items.astype(jnp.int32).reshape(_NW * _ROWS_PER_W, _CHUNK)
    flat = scores.reshape(-1)
    out = _GATHER(u2, it2, flat)
    return out.reshape(_BATCH)


# speedup vs baseline: 1.0967x; 1.0949x over previous
import functools

import jax
import jax.numpy as jnp
from jax import lax
from jax.experimental import pallas as pl
from jax.experimental.pallas import tpu as pltpu
from jax.experimental.pallas import tpu_sc as plsc

_N_USERS = 100000
_N_ITEMS = 1000
_PAD_ITEMS = 1024
_BATCH = 16384
_NC, _NS, _L = 2, 16, 16
_NW = _NC * _NS
_CHUNK = 128
_ROWS_PER_W = _BATCH // (_NW * _CHUNK)
_GROWS = 64


def _build():
    mesh = plsc.VectorSubcoreMesh(core_axis_name="c", subcore_axis_name="s")

    @functools.partial(
        pl.kernel,
        mesh=mesh,
        compiler_params=pltpu.CompilerParams(
            use_tc_tiling_on_sc=True, needs_layout_passes=False),
        out_type=jax.ShapeDtypeStruct((_NW * _ROWS_PER_W, _CHUNK), jnp.float32),
        scratch_types=[
            pltpu.VMEM((_ROWS_PER_W, _CHUNK), jnp.int32),
            pltpu.VMEM((_ROWS_PER_W, _CHUNK), jnp.int32),
            pltpu.VMEM((_GROWS, _PAD_ITEMS), jnp.float32),
            pltpu.VMEM((_ROWS_PER_W, _CHUNK), jnp.float32),
            pltpu.SemaphoreType.DMA,
        ],
    )
    def gather_kernel(users_hbm, items_hbm, scores_hbm, out_hbm,
                      u_v, it_v, rows_v, sel_v, sem):
        wid = lax.axis_index("s") * _NC + lax.axis_index("c")
        base = wid * _ROWS_PER_W
        pltpu.sync_copy(users_hbm.at[pl.ds(base, _ROWS_PER_W)], u_v)
        pltpu.sync_copy(items_hbm.at[pl.ds(base, _ROWS_PER_W)], it_v)
        sub = lax.iota(jnp.int32, _L)
        n_chunks = (_ROWS_PER_W * _CHUNK) // _GROWS
        per_row = _CHUNK // _GROWS
        for c in range(n_chunks):
            j, off = c // per_row, (c % per_row) * _GROWS
            pltpu.async_copy(
                scores_hbm.at[u_v.at[j, pl.ds(off, _GROWS)]], rows_v, sem
            ).wait()
            for g in range(_GROWS // _L):
                s = pl.ds(off + g * _L, _L)
                sel_v[j, s] = plsc.load_gather(
                    rows_v, [sub + g * _L, it_v[j, s]])
        pltpu.sync_copy(sel_v, out_hbm.at[pl.ds(base, _ROWS_PER_W)])

    return gather_kernel


_GATHER = _build()


def kernel(users, items, scores):
    u2 = users.astype(jnp.int32).reshape(_NW * _ROWS_PER_W, _CHUNK)
    it2 = items.astype(jnp.int32).reshape(_NW * _ROWS_PER_W, _CHUNK)
    padded = jnp.pad(scores, ((0, 0), (0, _PAD_ITEMS - _N_ITEMS)))
    out = _GATHER(u2, it2, padded)
    return out.reshape(_BATCH)
